# speedup vs baseline: 1.1467x; 1.1467x over previous
import math

import jax
import jax.numpy as jnp
import numpy as np
from jax.experimental import pallas as pl
from jax.experimental.pallas import tpu as pltpu

_SIN_EMB = 96
_TOK_EMB = 32
_OBS_VOCAB = 535
_ACT_VOCAB = 25
_VOCAB = _OBS_VOCAB + _ACT_VOCAB
_SCALE = 1000.0
_FEAT = 128
_BB = 64


def _lane_vectors():
    freqs = np.exp(np.arange(0, _SIN_EMB, 2, dtype=np.float32)
                   * (-math.log(10000.0) / _SIN_EMB))
    g = np.concatenate([np.zeros(_TOK_EMB, np.float32), freqs, freqs])
    p = np.concatenate([np.zeros(_TOK_EMB + 48, np.float32),
                        np.full(48, -math.pi / 2, np.float32)])
    return g.reshape(1, _FEAT), p.reshape(1, _FEAT)


def _enc_kernel(x_ref, idx_ref, tab_ref, g_ref, p_ref, out_ref, emb_ref):
    T = x_ref.shape[1]

    @pl.when(pl.program_id(0) == 0)
    def _gather():
        idx = idx_ref[...]
        iota = jax.lax.broadcasted_iota(jnp.int32, (T, _VOCAB), 1)
        onehot = (idx == iota).astype(jnp.float32)
        emb_ref[...] = jnp.dot(onehot, tab_ref[...],
                               preferred_element_type=jnp.float32)

    x = x_ref[...]
    g = g_ref[...]
    p = p_ref[...]
    phase = x[:, :, None] * g[None, :, :] + p[None, :, :]
    trig = jnp.cos(phase)
    lane = jax.lax.broadcasted_iota(jnp.int32, (1, T, _FEAT), 2)
    out_ref[...] = jnp.where(lane < _TOK_EMB, emb_ref[...][None, :, :], trig)


@jax.jit
def kernel(state_t, action_t, obs_table, act_table, obs_idx, act_idx):
    B, S = state_t.shape
    A = action_t.shape[1]
    T = S + A

    x_all = jnp.concatenate([state_t, action_t], axis=1) * _SCALE
    idx_all = jnp.concatenate(
        [obs_idx.astype(jnp.int32), act_idx.astype(jnp.int32) + _OBS_VOCAB]
    ).reshape(T, 1)
    tab = jnp.concatenate([obs_table, act_table], axis=0)
    tab_pad = jnp.pad(tab, ((0, 0), (0, _FEAT - _TOK_EMB)))
    g, p = _lane_vectors()

    grid = (B // _BB,)
    out = pl.pallas_call(
        _enc_kernel,
        grid=grid,
        in_specs=[
            pl.BlockSpec((_BB, T), lambda i: (i, 0)),
            pl.BlockSpec((T, 1), lambda i: (0, 0)),
            pl.BlockSpec((_VOCAB, _FEAT), lambda i: (0, 0)),
            pl.BlockSpec((1, _FEAT), lambda i: (0, 0)),
            pl.BlockSpec((1, _FEAT), lambda i: (0, 0)),
        ],
        out_specs=pl.BlockSpec((_BB, T, _FEAT), lambda i: (i, 0, 0)),
        out_shape=jax.ShapeDtypeStruct((B, T, _FEAT), jnp.float32),
        scratch_shapes=[pltpu.VMEM((T, _FEAT), jnp.float32)],
    )(x_all, idx_all, tab_pad, jnp.asarray(g), jnp.asarray(p))
    return out


# speedup vs baseline: 3.0667x; 2.6744x over previous
import math

import jax
import jax.numpy as jnp
import numpy as np
from jax.experimental import pallas as pl
from jax.experimental.pallas import tpu as pltpu

_SIN_EMB = 96
_TOK_EMB = 32
_OBS_VOCAB = 535
_ACT_VOCAB = 25
_VOCAB = _OBS_VOCAB + _ACT_VOCAB
_SCALE = 1000.0
_FEAT = 128
_BB = 64


def _lane_vectors():
    freqs = np.exp(np.arange(0, _SIN_EMB, 2, dtype=np.float32)
                   * (-math.log(10000.0) / _SIN_EMB))
    g = np.concatenate([np.zeros(_TOK_EMB, np.float32), freqs, freqs])
    p = np.concatenate([np.zeros(_TOK_EMB + 48, np.float32),
                        np.full(48, -math.pi / 2, np.float32)])
    return g.reshape(1, _FEAT), p.reshape(1, _FEAT)


_INV_2PI = 0.15915494309189535
_C1 = 6.28125
_C2 = 0.0019350051879882812
_C3 = 3.019916050561733e-07
_COS_COEF = (1.0, -0.49999988079071045, 0.04166647046804428,
             -0.0013887712266296148, 2.47679327003425e-05,
             -2.705998838337109e-07, 1.717589626082372e-09)


def _fast_cos(phase):
    k = jnp.round(phase * _INV_2PI)
    r = phase - k * _C1
    r = r - k * _C2
    r = r - k * _C3
    u = r * r
    acc = jnp.full_like(u, _COS_COEF[-1])
    for c in _COS_COEF[-2::-1]:
        acc = acc * u + c
    return acc


def _enc_kernel(x_ref, idx_ref, tab_ref, g_ref, p_ref, out_ref, emb_ref):
    T = x_ref.shape[1]

    @pl.when(pl.program_id(0) == 0)
    def _gather():
        idx = idx_ref[...]
        iota = jax.lax.broadcasted_iota(jnp.int32, (T, _VOCAB), 1)
        onehot = (idx == iota).astype(jnp.float32)
        emb_ref[...] = jnp.dot(onehot, tab_ref[...],
                               preferred_element_type=jnp.float32)

    x = x_ref[...]
    g = g_ref[...]
    p = p_ref[...]
    phase = x[:, :, None] * g[None, :, :] + p[None, :, :]
    trig = _fast_cos(phase)
    lane = jax.lax.broadcasted_iota(jnp.int32, (1, T, _FEAT), 2)
    out_ref[...] = jnp.where(lane < _TOK_EMB, emb_ref[...][None, :, :], trig)


@jax.jit
def kernel(state_t, action_t, obs_table, act_table, obs_idx, act_idx):
    B, S = state_t.shape
    A = action_t.shape[1]
    T = S + A

    x_all = jnp.concatenate([state_t, action_t], axis=1) * _SCALE
    idx_all = jnp.concatenate(
        [obs_idx.astype(jnp.int32), act_idx.astype(jnp.int32) + _OBS_VOCAB]
    ).reshape(T, 1)
    tab = jnp.concatenate([obs_table, act_table], axis=0)
    tab_pad = jnp.pad(tab, ((0, 0), (0, _FEAT - _TOK_EMB)))
    g, p = _lane_vectors()

    grid = (B // _BB,)
    out = pl.pallas_call(
        _enc_kernel,
        grid=grid,
        in_specs=[
            pl.BlockSpec((_BB, T), lambda i: (i, 0)),
            pl.BlockSpec((T, 1), lambda i: (0, 0)),
            pl.BlockSpec((_VOCAB, _FEAT), lambda i: (0, 0)),
            pl.BlockSpec((1, _FEAT), lambda i: (0, 0)),
            pl.BlockSpec((1, _FEAT), lambda i: (0, 0)),
        ],
        out_specs=pl.BlockSpec((_BB, T, _FEAT), lambda i: (i, 0, 0)),
        out_shape=jax.ShapeDtypeStruct((B, T, _FEAT), jnp.float32),
        scratch_shapes=[pltpu.VMEM((T, _FEAT), jnp.float32)],
    )(x_all, idx_all, tab_pad, jnp.asarray(g), jnp.asarray(p))
    return out


# speedup vs baseline: 3.7221x; 1.2137x over previous
import math

import jax
import jax.numpy as jnp
import numpy as np
from jax.experimental import pallas as pl
from jax.experimental.pallas import tpu as pltpu

_SIN_EMB = 96
_TOK_EMB = 32
_OBS_VOCAB = 535
_ACT_VOCAB = 25
_VOCAB = _OBS_VOCAB + _ACT_VOCAB
_SCALE = 1000.0
_FEAT = 128
_BB = 64


def _lane_vectors():
    freqs = np.exp(np.arange(0, _SIN_EMB, 2, dtype=np.float64)
                   * (-math.log(10000.0) / _SIN_EMB))
    turns = (_SCALE / (2.0 * math.pi)) * freqs
    g = np.concatenate([np.zeros(_TOK_EMB), turns, turns]).astype(np.float32)
    p = np.concatenate([np.zeros(_TOK_EMB + 48),
                        np.full(48, -0.25)]).astype(np.float32)
    return g.reshape(1, _FEAT), p.reshape(1, _FEAT)


_COS_COEF = (0.9999991059303284, -19.738954544067383, 64.92772674560547,
             -85.26000213623047, 58.73183059692383, -20.96858024597168)


def _fast_cos_turns(phase):
    f = phase - jnp.round(phase)
    u = f * f
    acc = jnp.full_like(u, _COS_COEF[-1])
    for c in _COS_COEF[-2::-1]:
        acc = acc * u + c
    return acc


def _enc_kernel(x_ref, idx_ref, tab_ref, g_ref, p_ref, out_ref, emb_ref):
    T = x_ref.shape[1]

    @pl.when(pl.program_id(0) == 0)
    def _gather():
        idx = idx_ref[...]
        iota = jax.lax.broadcasted_iota(jnp.int32, (T, _VOCAB), 1)
        onehot = (idx == iota).astype(jnp.float32)
        emb_ref[...] = jnp.dot(onehot, tab_ref[...],
                               preferred_element_type=jnp.float32)

    x = x_ref[...]
    g = g_ref[...]
    p = p_ref[...]
    phase = x[:, :, None] * g[None, :, :] + p[None, :, :]
    trig = _fast_cos_turns(phase)
    lane = jax.lax.broadcasted_iota(jnp.int32, (1, T, _FEAT), 2)
    out_ref[...] = jnp.where(lane < _TOK_EMB, emb_ref[...][None, :, :], trig)


@jax.jit
def kernel(state_t, action_t, obs_table, act_table, obs_idx, act_idx):
    B, S = state_t.shape
    A = action_t.shape[1]
    T = S + A

    x_all = jnp.concatenate([state_t, action_t], axis=1)
    idx_all = jnp.concatenate(
        [obs_idx.astype(jnp.int32), act_idx.astype(jnp.int32) + _OBS_VOCAB]
    ).reshape(T, 1)
    tab = jnp.concatenate([obs_table, act_table], axis=0)
    tab_pad = jnp.pad(tab, ((0, 0), (0, _FEAT - _TOK_EMB)))
    g, p = _lane_vectors()

    grid = (B // _BB,)
    out = pl.pallas_call(
        _enc_kernel,
        grid=grid,
        in_specs=[
            pl.BlockSpec((_BB, T), lambda i: (i, 0)),
            pl.BlockSpec((T, 1), lambda i: (0, 0)),
            pl.BlockSpec((_VOCAB, _FEAT), lambda i: (0, 0)),
            pl.BlockSpec((1, _FEAT), lambda i: (0, 0)),
            pl.BlockSpec((1, _FEAT), lambda i: (0, 0)),
        ],
        out_specs=pl.BlockSpec((_BB, T, _FEAT), lambda i: (i, 0, 0)),
        out_shape=jax.ShapeDtypeStruct((B, T, _FEAT), jnp.float32),
        scratch_shapes=[pltpu.VMEM((T, _FEAT), jnp.float32)],
    )(x_all, idx_all, tab_pad, jnp.asarray(g), jnp.asarray(p))
    return out


# speedup vs baseline: 8.0238x; 2.1557x over previous
import math

import jax
import jax.numpy as jnp
import numpy as np
from jax.experimental import pallas as pl
from jax.experimental.pallas import tpu as pltpu

_SIN_EMB = 96
_TOK_EMB = 32
_OBS_VOCAB = 535
_ACT_VOCAB = 25
_VOCAB = _OBS_VOCAB + _ACT_VOCAB
_SCALE = 1000.0
_FEAT = 128
_BB = 128


def _lane_vectors():
    freqs = np.exp(np.arange(0, _SIN_EMB, 2, dtype=np.float64)
                   * (-math.log(10000.0) / _SIN_EMB))
    turns = (_SCALE / (2.0 * math.pi)) * freqs
    g = np.concatenate([np.zeros(_TOK_EMB), turns, turns]).astype(np.float32)
    p = np.concatenate([np.zeros(_TOK_EMB + 48),
                        np.full(48, -0.25)]).astype(np.float32)
    return g.reshape(1, _FEAT), p.reshape(1, _FEAT)


_COS_COEF = (0.9999991059303284, -19.738954544067383, 64.92772674560547,
             -85.26000213623047, 58.73183059692383, -20.96858024597168)


def _fast_cos_turns(phase):
    f = phase - jnp.round(phase)
    u = f * f
    acc = jnp.full_like(u, _COS_COEF[-1])
    for c in _COS_COEF[-2::-1]:
        acc = acc * u + c
    return acc


def _enc_kernel(x_ref, idx_ref, tab_ref, g_ref, p_ref, out_ref, emb_ref):
    T = x_ref.shape[0]

    @pl.when(pl.program_id(0) == 0)
    def _gather():
        idx = idx_ref[...]
        iota = jax.lax.broadcasted_iota(jnp.int32, (T, _VOCAB), 1)
        onehot = (idx == iota).astype(jnp.float32)
        emb_ref[...] = jnp.dot(onehot, tab_ref[...],
                               preferred_element_type=jnp.float32)

    x = x_ref[...]
    g = g_ref[...]
    p = p_ref[...]
    phase = x[:, :, None] * g[None, :, :] + p[None, :, :]
    trig = _fast_cos_turns(phase)
    lane = jax.lax.broadcasted_iota(jnp.int32, (T, 1, _FEAT), 2)
    out_ref[...] = jnp.where(lane < _TOK_EMB, emb_ref[...][:, None, :], trig)


@jax.jit
def kernel(state_t, action_t, obs_table, act_table, obs_idx, act_idx):
    B, S = state_t.shape
    A = action_t.shape[1]
    T = S + A

    x_t = jnp.concatenate([state_t.T, action_t.T], axis=0)
    idx_all = jnp.concatenate(
        [obs_idx.astype(jnp.int32), act_idx.astype(jnp.int32) + _OBS_VOCAB]
    ).reshape(T, 1)
    tab = jnp.concatenate([obs_table, act_table], axis=0)
    tab_pad = jnp.pad(tab, ((0, 0), (0, _FEAT - _TOK_EMB)))
    g, p = _lane_vectors()

    grid = (B // _BB,)
    out = pl.pallas_call(
        _enc_kernel,
        grid=grid,
        in_specs=[
            pl.BlockSpec((T, _BB), lambda i: (0, i)),
            pl.BlockSpec((T, 1), lambda i: (0, 0)),
            pl.BlockSpec((_VOCAB, _FEAT), lambda i: (0, 0)),
            pl.BlockSpec((1, _FEAT), lambda i: (0, 0)),
            pl.BlockSpec((1, _FEAT), lambda i: (0, 0)),
        ],
        out_specs=pl.BlockSpec((T, _BB, _FEAT), lambda i: (0, i, 0)),
        out_shape=jax.ShapeDtypeStruct((T, B, _FEAT), jnp.float32),
        scratch_shapes=[pltpu.VMEM((T, _FEAT), jnp.float32)],
    )(x_t, idx_all, tab_pad, jnp.asarray(g), jnp.asarray(p))
    return jnp.transpose(out, (1, 0, 2))


# speedup vs baseline: 9.0410x; 1.1268x over previous
import math

import jax
import jax.numpy as jnp
import numpy as np
from jax.experimental import pallas as pl
from jax.experimental.pallas import tpu as pltpu

_SIN_EMB = 96
_TOK_EMB = 32
_OBS_VOCAB = 535
_ACT_VOCAB = 25
_VOCAB = _OBS_VOCAB + _ACT_VOCAB
_SCALE = 1000.0
_FEAT = 128
_BB = 128


def _lane_vectors():
    freqs = np.exp(np.arange(0, _SIN_EMB, 2, dtype=np.float64)
                   * (-math.log(10000.0) / _SIN_EMB))
    turns = (_SCALE / (2.0 * math.pi)) * freqs
    g = np.concatenate([np.zeros(_TOK_EMB), turns, turns]).astype(np.float32)
    p = np.concatenate([np.zeros(_TOK_EMB + 48),
                        np.full(48, -0.25)]).astype(np.float32)
    return g.reshape(1, _FEAT), p.reshape(1, _FEAT)


_COS_COEF = (0.9999540448188782, -19.730182647705078, 64.65386199951172,
             -82.26329040527344, 45.33661651611328)


def _fast_cos_turns(phase):
    f = phase - jnp.round(phase)
    u = f * f
    acc = jnp.full_like(u, _COS_COEF[-1])
    for c in _COS_COEF[-2::-1]:
        acc = acc * u + c
    return acc


def _enc_kernel(x_ref, idx_ref, tab_ref, g_ref, p_ref, out_ref, emb_ref):
    T = x_ref.shape[0]

    @pl.when(pl.program_id(0) == 0)
    def _gather():
        idx = idx_ref[...]
        iota = jax.lax.broadcasted_iota(jnp.int32, (T, _VOCAB), 1)
        onehot = (idx == iota).astype(jnp.float32)
        emb_ref[...] = jnp.dot(onehot, tab_ref[...],
                               preferred_element_type=jnp.float32)

    x = x_ref[...]
    g = g_ref[...]
    p = p_ref[...]
    phase = x[:, :, None] * g[None, :, :] + p[None, :, :]
    trig = _fast_cos_turns(phase)
    lane = jax.lax.broadcasted_iota(jnp.int32, (T, 1, _FEAT), 2)
    out_ref[...] = jnp.where(lane < _TOK_EMB, emb_ref[...][:, None, :], trig)


@jax.jit
def kernel(state_t, action_t, obs_table, act_table, obs_idx, act_idx):
    B, S = state_t.shape
    A = action_t.shape[1]
    T = S + A

    x_t = jnp.concatenate([state_t.T, action_t.T], axis=0)
    idx_all = jnp.concatenate(
        [obs_idx.astype(jnp.int32), act_idx.astype(jnp.int32) + _OBS_VOCAB]
    ).reshape(T, 1)
    tab = jnp.concatenate([obs_table, act_table], axis=0)
    tab_pad = jnp.pad(tab, ((0, 0), (0, _FEAT - _TOK_EMB)))
    g, p = _lane_vectors()

    grid = (B // _BB,)
    out = pl.pallas_call(
        _enc_kernel,
        grid=grid,
        in_specs=[
            pl.BlockSpec((T, _BB), lambda i: (0, i)),
            pl.BlockSpec((T, 1), lambda i: (0, 0)),
            pl.BlockSpec((_VOCAB, _FEAT), lambda i: (0, 0)),
            pl.BlockSpec((1, _FEAT), lambda i: (0, 0)),
            pl.BlockSpec((1, _FEAT), lambda i: (0, 0)),
        ],
        out_specs=pl.BlockSpec((T, _BB, _FEAT), lambda i: (0, i, 0)),
        out_shape=jax.ShapeDtypeStruct((T, B, _FEAT), jnp.float32),
        scratch_shapes=[pltpu.VMEM((T, _FEAT), jnp.float32)],
    )(x_t, idx_all, tab_pad, jnp.asarray(g), jnp.asarray(p))
    return jnp.transpose(out, (1, 0, 2))


# speedup vs baseline: 10.2664x; 1.1355x over previous
import math

import jax
import jax.numpy as jnp
import numpy as np
from jax.experimental import pallas as pl
from jax.experimental.pallas import tpu as pltpu

_SIN_EMB = 96
_TOK_EMB = 32
_OBS_VOCAB = 535
_ACT_VOCAB = 25
_VOCAB = _OBS_VOCAB + _ACT_VOCAB
_SCALE = 1000.0
_FEAT = 128
_BB = 256


def _lane_vectors():
    freqs = np.exp(np.arange(0, _SIN_EMB, 2, dtype=np.float64)
                   * (-math.log(10000.0) / _SIN_EMB))
    turns = (_SCALE / (2.0 * math.pi)) * freqs
    g = np.concatenate([np.zeros(_TOK_EMB), turns, turns]).astype(np.float32)
    p = np.concatenate([np.zeros(_TOK_EMB + 48),
                        np.full(48, -0.25)]).astype(np.float32)
    return g.reshape(1, _FEAT), p.reshape(1, _FEAT)


_COS_COEF = (0.9984107613563538, -19.539045333862305, 60.93540954589844,
             -59.054115295410156)


def _fast_cos_turns(phase):
    f = phase - jnp.round(phase)
    u = f * f
    acc = jnp.full_like(u, _COS_COEF[-1])
    for c in _COS_COEF[-2::-1]:
        acc = acc * u + c
    return acc


def _enc_kernel(x_ref, idx_ref, tab_ref, g_ref, p_ref, out_ref, emb_ref):
    T = x_ref.shape[0]

    @pl.when(pl.program_id(0) == 0)
    def _gather():
        idx = idx_ref[...]
        iota = jax.lax.broadcasted_iota(jnp.int32, (T, _VOCAB), 1)
        onehot = (idx == iota).astype(jnp.float32)
        emb_ref[...] = jnp.dot(onehot, tab_ref[...],
                               preferred_element_type=jnp.float32)

    x = x_ref[...]
    g = g_ref[...]
    p = p_ref[...]
    phase = x[:, :, None] * g[None, :, :] + p[None, :, :]
    trig = _fast_cos_turns(phase)
    lane = jax.lax.broadcasted_iota(jnp.int32, (T, 1, _FEAT), 2)
    out_ref[...] = jnp.where(lane < _TOK_EMB, emb_ref[...][:, None, :], trig)


@jax.jit
def kernel(state_t, action_t, obs_table, act_table, obs_idx, act_idx):
    B, S = state_t.shape
    A = action_t.shape[1]
    T = S + A

    x_t = jnp.concatenate([state_t.T, action_t.T], axis=0)
    idx_all = jnp.concatenate(
        [obs_idx.astype(jnp.int32), act_idx.astype(jnp.int32) + _OBS_VOCAB]
    ).reshape(T, 1)
    tab = jnp.concatenate([obs_table, act_table], axis=0)
    tab_pad = jnp.pad(tab, ((0, 0), (0, _FEAT - _TOK_EMB)))
    g, p = _lane_vectors()

    grid = (B // _BB,)
    out = pl.pallas_call(
        _enc_kernel,
        grid=grid,
        in_specs=[
            pl.BlockSpec((T, _BB), lambda i: (0, i)),
            pl.BlockSpec((T, 1), lambda i: (0, 0)),
            pl.BlockSpec((_VOCAB, _FEAT), lambda i: (0, 0)),
            pl.BlockSpec((1, _FEAT), lambda i: (0, 0)),
            pl.BlockSpec((1, _FEAT), lambda i: (0, 0)),
        ],
        out_specs=pl.BlockSpec((T, _BB, _FEAT), lambda i: (0, i, 0)),
        out_shape=jax.ShapeDtypeStruct((T, B, _FEAT), jnp.float32),
        scratch_shapes=[pltpu.VMEM((T, _FEAT), jnp.float32)],
    )(x_t, idx_all, tab_pad, jnp.asarray(g), jnp.asarray(p))
    return jnp.transpose(out, (1, 0, 2))


# speedup vs baseline: 10.9534x; 1.0669x over previous
import math

import jax
import jax.numpy as jnp
import numpy as np
from jax.experimental import pallas as pl
from jax.experimental.pallas import tpu as pltpu

_SIN_EMB = 96
_TOK_EMB = 32
_OBS_VOCAB = 535
_ACT_VOCAB = 25
_VOCAB = _OBS_VOCAB + _ACT_VOCAB
_SCALE = 1000.0
_FEAT = 128
_BB = 256


def _lane_vectors():
    freqs = np.exp(np.arange(0, _SIN_EMB, 2, dtype=np.float64)
                   * (-math.log(10000.0) / _SIN_EMB))
    turns = (_SCALE / (2.0 * math.pi)) * freqs
    g = np.concatenate([np.zeros(_TOK_EMB), turns, turns]).astype(np.float32)
    p = np.concatenate([np.zeros(_TOK_EMB + 48),
                        np.full(48, -0.25)]).astype(np.float32)
    return g.reshape(1, _FEAT), p.reshape(1, _FEAT)


_COS_COEF = (0.9984107613563538, -19.539045333862305, 60.93540954589844,
             -59.054115295410156)


def _fast_cos_turns(phase):
    f = phase - jnp.round(phase)
    u = f * f
    acc = jnp.full_like(u, _COS_COEF[-1])
    for c in _COS_COEF[-2::-1]:
        acc = acc * u + c
    return acc


_INV_2PI = float(1.0 / (2.0 * math.pi))


def _enc_kernel(x_ref, idx_ref, tab_ref, g_ref, p_ref, out_ref, pha_ref):
    T = x_ref.shape[0]

    @pl.when(pl.program_id(0) == 0)
    def _gather():
        idx = idx_ref[...]
        iota = jax.lax.broadcasted_iota(jnp.int32, (T, _VOCAB), 1)
        onehot = (idx == iota).astype(jnp.float32)
        v = jnp.dot(onehot, tab_ref[...],
                    preferred_element_type=jnp.float32)
        asin = v * (1.0 + (1.0 / 6.0) * v * v)
        p_emb = 0.25 - asin * _INV_2PI
        lane = jax.lax.broadcasted_iota(jnp.int32, (T, _FEAT), 1)
        pha_ref[...] = jnp.where(lane < _TOK_EMB, p_emb,
                                 jnp.broadcast_to(p_ref[...], (T, _FEAT)))

    x = x_ref[...]
    g = g_ref[...]
    phase = x[:, :, None] * g[None, :, :] + pha_ref[...][:, None, :]
    out_ref[...] = _fast_cos_turns(phase)


@jax.jit
def kernel(state_t, action_t, obs_table, act_table, obs_idx, act_idx):
    B, S = state_t.shape
    A = action_t.shape[1]
    T = S + A

    x_t = jnp.concatenate([state_t.T, action_t.T], axis=0)
    idx_all = jnp.concatenate(
        [obs_idx.astype(jnp.int32), act_idx.astype(jnp.int32) + _OBS_VOCAB]
    ).reshape(T, 1)
    tab = jnp.concatenate([obs_table, act_table], axis=0)
    tab_pad = jnp.pad(tab, ((0, 0), (0, _FEAT - _TOK_EMB)))
    g, p = _lane_vectors()

    grid = (B // _BB,)
    out = pl.pallas_call(
        _enc_kernel,
        grid=grid,
        in_specs=[
            pl.BlockSpec((T, _BB), lambda i: (0, i)),
            pl.BlockSpec((T, 1), lambda i: (0, 0)),
            pl.BlockSpec((_VOCAB, _FEAT), lambda i: (0, 0)),
            pl.BlockSpec((1, _FEAT), lambda i: (0, 0)),
            pl.BlockSpec((1, _FEAT), lambda i: (0, 0)),
        ],
        out_specs=pl.BlockSpec((T, _BB, _FEAT), lambda i: (0, i, 0)),
        out_shape=jax.ShapeDtypeStruct((T, B, _FEAT), jnp.float32),
        scratch_shapes=[pltpu.VMEM((T, _FEAT), jnp.float32)],
    )(x_t, idx_all, tab_pad, jnp.asarray(g), jnp.asarray(p))
    return jnp.transpose(out, (1, 0, 2))
